# baseline (speedup 1.0000x reference)
import functools

import jax
import jax.numpy as jnp
from jax import lax
from jax.experimental import pallas as pl
from jax.experimental.pallas import tpu as pltpu
from jax.experimental.pallas import tpu_sc as plsc

VOCAB = 100000
DIM = 64
BATCH = 4096
SEQ = 200

NC = 2
NS = 16
NW = NC * NS
ROWS_W = BATCH // NW

IDXM = 100
CHB = 2
NG = CHB * SEQ // IDXM
NBUF = 4
G = ROWS_W // CHB // NBUF

_mesh = plsc.VectorSubcoreMesh(
    core_axis_name="c", subcore_axis_name="s", num_cores=NC, num_subcores=NS
)


@functools.partial(
    pl.kernel,
    out_type=jax.ShapeDtypeStruct((BATCH, SEQ, DIM), jnp.float32),
    mesh=_mesh,
    scratch_types=[
        pltpu.VMEM((NBUF, NG, IDXM), jnp.int32),
        pltpu.VMEM((NBUF, CHB, SEQ, DIM), jnp.float32),
        pltpu.SemaphoreType.DMA,
        pltpu.SemaphoreType.DMA,
        pltpu.SemaphoreType.DMA,
    ],
    compiler_params=pltpu.CompilerParams(use_tc_tiling_on_sc=False),
)
def _sc_gather(x_hbm, tab_hbm, out_hbm, idx_v, rows_v, sem_i, sem_g, sem_w):
    wid = lax.axis_index("s") * NC + lax.axis_index("c")
    base_b = wid * ROWS_W

    def idx_src(i):
        return x_hbm.at[pl.ds((base_b + i * CHB) * SEQ // IDXM, NG)]

    def out_dst(i):
        return out_hbm.at[pl.ds(base_b + i * CHB, CHB)]

    def gather_cp(b, j):
        return pltpu.make_async_copy(
            tab_hbm.at[idx_v.at[b, j]],
            rows_v.at[b, j // 2, pl.ds((j % 2) * IDXM, IDXM)],
            sem_g,
        )

    for b in range(NBUF):
        pltpu.async_copy(idx_src(b), idx_v.at[b], sem_i)

    def outer(g, carry):
        i0 = g * NBUF
        for b in range(NBUF):
            i = i0 + b
            pltpu.make_async_copy(idx_src(i), idx_v.at[b], sem_i).wait()

            @pl.when(g > 0)
            def _():
                pltpu.make_async_copy(rows_v.at[b], out_dst(i), sem_w).wait()

            for j in range(NG):
                gather_cp(b, j).start()
        for b in range(NBUF):
            i = i0 + b
            for j in range(NG):
                gather_cp(b, j).wait()
            pltpu.async_copy(rows_v.at[b], out_dst(i), sem_w)

            @pl.when(g < G - 1)
            def _():
                pltpu.async_copy(idx_src(i + NBUF), idx_v.at[b], sem_i)

        return carry

    lax.fori_loop(0, G, outer, 0)

    for b in range(NBUF):
        pltpu.make_async_copy(
            rows_v.at[b], out_dst((G - 1) * NBUF + b), sem_w
        ).wait()


def _mask_body(x_ref, o_ref):
    o_ref[...] = x_ref[...] != 0


_tc_mask = pl.pallas_call(
    _mask_body,
    out_shape=jax.ShapeDtypeStruct((BATCH, SEQ), jnp.bool_),
    grid=(BATCH // 512,),
    in_specs=[pl.BlockSpec((512, SEQ), lambda i: (i, 0))],
    out_specs=pl.BlockSpec((512, SEQ), lambda i: (i, 0)),
)


def kernel(x, table, concepts):
    ext = jnp.concatenate([table, concepts], axis=0)
    xf = x.reshape(BATCH * SEQ // IDXM, IDXM)
    embeds = _sc_gather(xf, ext)
    mask = _tc_mask(x)
    return embeds, mask
